# baseline (speedup 1.0000x reference)
import functools

import jax
import jax.numpy as jnp
from jax import lax
from jax.experimental import pallas as pl
from jax.experimental.pallas import tpu as pltpu
from jax.experimental.pallas import tpu_sc as plsc

N = 800000
D = 64
NP = N // 2
T2_ROWS = 3600

NC = 2
NS = 16
NW = NC * NS
C = 128
NB = 4
NCHT = NP // C
ITERS = -(-NCHT // NW)
GP = C // 16
SPLITS = ((0, 128),)

_mesh = plsc.VectorSubcoreMesh(core_axis_name="c", subcore_axis_name="s")


@functools.partial(
    pl.kernel,
    out_type=jax.ShapeDtypeStruct((NP, 2 * D), jnp.float32),
    mesh=_mesh,
    scratch_types=[
        [[pltpu.VMEM((C,), jnp.int32) for _ in range(6)] for _ in range(NB)],
        [pltpu.VMEM((C,), jnp.int32) for _ in range(NB)],
        [pltpu.VMEM((C, 2 * D), jnp.float32) for _ in range(NB)],
        [pltpu.SemaphoreType.DMA for _ in range(NB)],
        [pltpu.SemaphoreType.DMA for _ in range(NB)],
        [pltpu.SemaphoreType.DMA for _ in range(NB)],
    ],
)
def _bond_encode(ea0_hbm, ea1_hbm, ea2_hbm, t2_hbm, out_hbm,
                 attr, keys, rows, sa, sg, so):
    ea_cols = (ea0_hbm, ea1_hbm, ea2_hbm)
    wid = lax.axis_index("s") * NC + lax.axis_index("c")

    def cid_of(i):
        return wid + i * NW

    def start_attr(i, b):
        r0 = cid_of(i) * C
        for k in range(3):
            pltpu.async_copy(ea_cols[k].at[pl.ds(r0, C)],
                             attr[b][k], sa[b])
            pltpu.async_copy(ea_cols[k].at[pl.ds(NP + r0, C)],
                             attr[b][3 + k], sa[b])

    def wait_attr(b):
        for k in range(6):
            pltpu.make_async_copy(ea0_hbm.at[pl.ds(0, C)],
                                  attr[b][k], sa[b]).wait()

    def fuse_keys(b):
        a = attr[b]
        v = keys[b]
        for g in range(GP):
            s = pl.ds(g * 16, 16)
            ie = a[0][s] * 12 + a[1][s] * 2 + a[2][s]
            io = a[3][s] * 12 + a[4][s] * 2 + a[5][s]
            v[s] = ie * 60 + io

    def start_gather(b):
        for off, cnt in SPLITS:
            pltpu.async_copy(
                t2_hbm.at[keys[b].at[pl.ds(off, cnt)]],
                rows[b].at[pl.ds(off, cnt)],
                sg[b],
            )

    def wait_gather(b):
        for off, cnt in SPLITS:
            pltpu.make_async_copy(
                t2_hbm.at[pl.ds(0, cnt)],
                rows[b].at[pl.ds(off, cnt)],
                sg[b],
            ).wait()

    def start_out(i, b):
        pltpu.async_copy(rows[b], out_hbm.at[pl.ds(cid_of(i) * C, C)], so[b])

    def wait_out(b):
        pltpu.make_async_copy(rows[b], out_hbm.at[pl.ds(0, C)], so[b]).wait()

    for b0 in range(NB):
        start_attr(b0, b0)

    def super_body(sv):
        for b in range(NB):
            i = sv * NB + b

            @pl.when((i < ITERS) & (cid_of(i) < NCHT))
            def _():
                wait_attr(b)
                fuse_keys(b)

                @pl.when(i >= NB)
                def _():
                    wait_out(b)

                start_gather(b)

                @pl.when((i + NB < ITERS) & (cid_of(i + NB) < NCHT))
                def _():
                    start_attr(i + NB, b)

            @pl.when((i >= 1) & (cid_of(i - 1) < NCHT))
            def _():
                wait_gather((b - 1) % NB)
                start_out(i - 1, (b - 1) % NB)

    pl.loop(0, (ITERS + NB) // NB)(super_body)

    for j in range(ITERS - NB, ITERS):
        @pl.when(cid_of(j) < NCHT)
        def _():
            wait_out(j % NB)


RP_B = 3200
RP_HALF = NP // RP_B


def _repack_body(x_ref, o_ref):
    h = pl.program_id(0)
    x = x_ref[...]
    half = jnp.where(h < RP_HALF, x[:, :D], x[:, D:])
    o_ref[...] = half.T


_repack = pl.pallas_call(
    _repack_body,
    grid=(2 * RP_HALF,),
    in_specs=[pl.BlockSpec((RP_B, 2 * D), lambda h: (h % RP_HALF, 0))],
    out_specs=pl.BlockSpec((D, RP_B), lambda h: (0, h)),
    out_shape=jax.ShapeDtypeStruct((D, N), jnp.float32),
)


def kernel(edge_attr, W0, W1, W2):
    t = (W0[:, None, None, :] + W1[None, :, None, :] + W2[None, None, :, :])
    t = t.reshape(60, D)
    t2 = jnp.concatenate(
        [jnp.broadcast_to(t[:, None, :], (60, 60, D)),
         jnp.broadcast_to(t[None, :, :], (60, 60, D))], axis=-1,
    ).reshape(T2_ROWS, 2 * D)
    out2 = _bond_encode(edge_attr[:, 0], edge_attr[:, 1], edge_attr[:, 2], t2)
    return _repack(out2).T


# speedup vs baseline: 1.0426x; 1.0426x over previous
import functools

import jax
import jax.numpy as jnp
from jax import lax
from jax.experimental import pallas as pl
from jax.experimental.pallas import tpu as pltpu
from jax.experimental.pallas import tpu_sc as plsc

N = 800000
D = 64
NP = N // 2
T2_ROWS = 3600

NC = 2
NS = 16
NW = NC * NS
C = 128
NB = 4
NCHT = NP // C
ITERS = -(-NCHT // NW)
GP = C // 16
SPLITS = ((0, 128),)

_mesh = plsc.VectorSubcoreMesh(core_axis_name="c", subcore_axis_name="s")


@functools.partial(
    pl.kernel,
    out_type=jax.ShapeDtypeStruct((NP, 2 * D), jnp.float32),
    mesh=_mesh,
    scratch_types=[
        [[pltpu.VMEM((C,), jnp.int32) for _ in range(6)] for _ in range(NB)],
        [pltpu.VMEM((C,), jnp.int32) for _ in range(NB)],
        [pltpu.VMEM((C, 2 * D), jnp.float32) for _ in range(NB)],
        [pltpu.SemaphoreType.DMA for _ in range(NB)],
        [pltpu.SemaphoreType.DMA for _ in range(NB)],
        [pltpu.SemaphoreType.DMA for _ in range(NB)],
    ],
)
def _bond_encode(ea0_hbm, ea1_hbm, ea2_hbm, t2_hbm, out_hbm,
                 attr, keys, rows, sa, sg, so):
    ea_cols = (ea0_hbm, ea1_hbm, ea2_hbm)
    wid = lax.axis_index("s") * NC + lax.axis_index("c")

    def cid_of(i):
        return wid + i * NW

    def start_attr(i, b):
        r0 = cid_of(i) * C
        for k in range(3):
            pltpu.async_copy(ea_cols[k].at[pl.ds(r0, C)],
                             attr[b][k], sa[b])
            pltpu.async_copy(ea_cols[k].at[pl.ds(NP + r0, C)],
                             attr[b][3 + k], sa[b])

    def wait_attr(b):
        for k in range(6):
            pltpu.make_async_copy(ea0_hbm.at[pl.ds(0, C)],
                                  attr[b][k], sa[b]).wait()

    def fuse_keys(b):
        a = attr[b]
        v = keys[b]
        for g in range(GP):
            s = pl.ds(g * 16, 16)
            ie = a[0][s] * 12 + a[1][s] * 2 + a[2][s]
            io = a[3][s] * 12 + a[4][s] * 2 + a[5][s]
            v[s] = ie * 60 + io

    def start_gather(b):
        for off, cnt in SPLITS:
            pltpu.async_copy(
                t2_hbm.at[keys[b].at[pl.ds(off, cnt)]],
                rows[b].at[pl.ds(off, cnt)],
                sg[b],
            )

    def wait_gather(b):
        for off, cnt in SPLITS:
            pltpu.make_async_copy(
                t2_hbm.at[pl.ds(0, cnt)],
                rows[b].at[pl.ds(off, cnt)],
                sg[b],
            ).wait()

    def start_out(i, b):
        pltpu.async_copy(rows[b], out_hbm.at[pl.ds(cid_of(i) * C, C)], so[b])

    def wait_out(b):
        pltpu.make_async_copy(rows[b], out_hbm.at[pl.ds(0, C)], so[b]).wait()

    for b0 in range(NB):
        start_attr(b0, b0)

    def super_body(sv):
        for b in range(NB):
            i = sv * NB + b

            @pl.when((i < ITERS) & (cid_of(i) < NCHT))
            def _():
                wait_attr(b)
                fuse_keys(b)

                @pl.when(i >= NB)
                def _():
                    wait_out(b)

                start_gather(b)

                @pl.when((i + NB < ITERS) & (cid_of(i + NB) < NCHT))
                def _():
                    start_attr(i + NB, b)

            @pl.when((i >= 1) & (cid_of(i - 1) < NCHT))
            def _():
                wait_gather((b - 1) % NB)
                start_out(i - 1, (b - 1) % NB)

    pl.loop(0, (ITERS + NB) // NB)(super_body)

    for j in range(ITERS - NB, ITERS):
        @pl.when(cid_of(j) < NCHT)
        def _():
            wait_out(j % NB)


RP_B = 3200
RP_HALF = NP // RP_B


def _repack_body(x_ref, o_ref):
    h = pl.program_id(0)
    x = x_ref[...]
    half = jnp.where(h < RP_HALF, x[:, :D], x[:, D:])
    eye = (lax.broadcasted_iota(jnp.int32, (D, D), 0)
           == lax.broadcasted_iota(jnp.int32, (D, D), 1)).astype(jnp.float32)
    o_ref[...] = jax.lax.dot_general(
        eye, half, (((1,), (1,)), ((), ())),
        preferred_element_type=jnp.float32,
    )


_repack = pl.pallas_call(
    _repack_body,
    grid=(2 * RP_HALF,),
    in_specs=[pl.BlockSpec((RP_B, 2 * D), lambda h: (h % RP_HALF, 0))],
    out_specs=pl.BlockSpec((D, RP_B), lambda h: (0, h)),
    out_shape=jax.ShapeDtypeStruct((D, N), jnp.float32),
)


def kernel(edge_attr, W0, W1, W2):
    t = (W0[:, None, None, :] + W1[None, :, None, :] + W2[None, None, :, :])
    t = t.reshape(60, D)
    t2 = jnp.concatenate(
        [jnp.broadcast_to(t[:, None, :], (60, 60, D)),
         jnp.broadcast_to(t[None, :, :], (60, 60, D))], axis=-1,
    ).reshape(T2_ROWS, 2 * D)
    out2 = _bond_encode(edge_attr[:, 0], edge_attr[:, 1], edge_attr[:, 2], t2)
    return _repack(out2).T
